# baseline (speedup 1.0000x reference)
import functools

import jax
import jax.numpy as jnp
from jax import lax
from jax.experimental import pallas as pl
from jax.experimental.pallas import tpu as pltpu
from jax.experimental.pallas import tpu_sc as plsc

L = 16
NC, NS = 2, 16
NW = NC * NS

RI = 24
RO = 2 * RI
NK = 9
HT = 16
EDGE = 7168


def _zero_weave_sc(x3, ztile, *, BC, H, W):
    Ho, Wo = 2 * H - 1, 2 * W - 1
    P = Ho * Wo
    CW = RO * Wo
    ch_per = BC // NW

    mesh = plsc.VectorSubcoreMesh(
        core_axis_name="c", subcore_axis_name="s", num_cores=NC, num_subcores=NS
    )

    @functools.partial(
        pl.kernel,
        out_type=jax.ShapeDtypeStruct((BC * P,), jnp.float32),
        mesh=mesh,
        scratch_types=[
            pltpu.VMEM((RI, W), jnp.float32),
            pltpu.VMEM((RI, W), jnp.float32),
            pltpu.VMEM((HT, W), jnp.float32),
            pltpu.VMEM((HT, W), jnp.float32),
            pltpu.VMEM((L,), jnp.float32),
            pltpu.VMEM((L,), jnp.float32),
            pltpu.VMEM((CW,), jnp.float32),
            pltpu.VMEM((CW,), jnp.float32),
            pltpu.VMEM((CW,), jnp.float32),
            pltpu.VMEM((EDGE,), jnp.float32),
            pltpu.VMEM((EDGE,), jnp.float32),
            pltpu.SemaphoreType.DMA,
            pltpu.SemaphoreType.DMA,
            pltpu.SemaphoreType.DMA,
            pltpu.SemaphoreType.DMA,
            pltpu.SemaphoreType.DMA,
            pltpu.SemaphoreType.DMA,
            pltpu.SemaphoreType.DMA,
            pltpu.SemaphoreType.DMA,
            pltpu.SemaphoreType.DMA,
            pltpu.SemaphoreType.DMA,
            pltpu.SemaphoreType.DMA,
        ],
        compiler_params=pltpu.CompilerParams(
            use_tc_tiling_on_sc=False, needs_layout_passes=False
        ),
    )
    def zw(x_hbm, z_hbm, out_hbm, ib0, ib1, ihb, itb, pvb, nxb,
           obA, obB, obC, ohb, otb,
           si0, si1, sih, sit, sp, sn, soA, soB, soC, soh, sot):
        wid = lax.axis_index("s") * NC + lax.axis_index("c")
        p = wid & 1
        idx16 = wid >> 1

        in_bufs, in_sems = (ib0, ib1), (si0, si1)
        out_bufs, out_sems = (obA, obB, obC), (soA, soB, soC)

        for ob, osem in zip(out_bufs, out_sems):
            pltpu.async_copy(z_hbm, ob, osem)
        pltpu.async_copy(z_hbm.at[pl.ds(0, EDGE)], ohb, soh)
        pltpu.async_copy(z_hbm.at[pl.ds(0, EDGE)], otb, sot)

        iota = lax.iota(jnp.int32, L)
        cvecs = [2 * (k * L + iota) for k in range(W // L)]
        iota2 = 2 * iota

        def scatter_block(ib, ob, n_rows, row0_in, base, stride):
            def do_row(m, c2):
                rb = lax.broadcast(base + stride * m, (L,))
                for k in range(W // L):
                    vals = ib[row0_in + m, pl.ds(k * L, L)]
                    plsc.store_scatter(ob, [rb + cvecs[k]], vals)
                return c2
            lax.fori_loop(0, n_rows, do_row, 0)

        def do_plane(ci, carry):
            ch = 2 * (idx16 * ch_per + ci) + p
            s = ch & 15
            a = s + 1
            w = ch * P
            i0 = (s + p) >> 1

            pltpu.async_copy(x_hbm.at[ch, pl.ds(0, HT), :], ihb, sih)
            pltpu.async_copy(x_hbm.at[ch, pl.ds(H - HT, HT), :], itb, sit)
            chm = jnp.maximum(ch - 1, 0)
            chp = jnp.minimum(ch + 1, BC - 1)
            pltpu.async_copy(x_hbm.at[chm, H - 1, pl.ds(W - L, L)], pvb, sp)
            pltpu.async_copy(x_hbm.at[chp, 0, pl.ds(0, L)], nxb, sn)
            pltpu.async_copy(x_hbm.at[ch, pl.ds(i0, RI), :], ib0, si0)
            pltpu.async_copy(x_hbm.at[ch, pl.ds(i0 + RI, RI), :], ib1, si1)

            pltpu.make_async_copy(x_hbm.at[ch, pl.ds(0, HT), :], ihb, sih).wait()
            pltpu.make_async_copy(x_hbm.at[chm, H - 1, pl.ds(W - L, L)], pvb, sp).wait()
            pltpu.make_async_copy(z_hbm.at[pl.ds(0, EDGE)], ohb, soh).wait()
            scatter_block(ihb, ohb, HT // 2, 0, s, 2 * Wo)
            idxp = lax.broadcast(s - 31, (L,)) + iota2
            plsc.store_scatter(ohb, [idxp], pvb[...], mask=idxp >= 0)
            idxq = lax.broadcast(HT * Wo + s, (L,)) + iota2
            plsc.store_scatter(ohb, [idxq], ihb[HT // 2, pl.ds(0, L)],
                               mask=idxq < EDGE)
            pltpu.async_copy(
                ohb, out_hbm.at[pl.ds(pl.multiple_of(w - s, 16), EDGE)], soh
            )

            for k in range(NK):
                qi, qo = k % 2, k % 3
                pltpu.make_async_copy(
                    x_hbm.at[ch, pl.ds(i0 + k * RI, RI), :],
                    in_bufs[qi], in_sems[qi],
                ).wait()
                pltpu.make_async_copy(z_hbm, out_bufs[qo], out_sems[qo]).wait()
                scatter_block(in_bufs[qi], out_bufs[qo], RI, 0, Wo * p, 2 * Wo)
                pltpu.async_copy(
                    out_bufs[qo],
                    out_hbm.at[pl.ds(
                        pl.multiple_of(w + Wo * s + CW * k, 16), CW)],
                    out_sems[qo],
                )
                if k + 2 < NK:
                    pltpu.async_copy(
                        x_hbm.at[ch, pl.ds(i0 + (k + 2) * RI, RI), :],
                        in_bufs[qi], in_sems[qi],
                    )

            pltpu.make_async_copy(x_hbm.at[ch, pl.ds(H - HT, HT), :], itb, sit).wait()
            pltpu.make_async_copy(x_hbm.at[chp, 0, pl.ds(0, L)], nxb, sn).wait()
            pltpu.make_async_copy(z_hbm.at[pl.ds(0, EDGE)], otb, sot).wait()
            scatter_block(itb, otb, HT // 2, HT // 2, Wo + a, 2 * Wo)
            idxr = lax.broadcast(a - 31, (L,)) + iota2
            plsc.store_scatter(otb, [idxr], itb[HT // 2 - 1, pl.ds(W - L, L)],
                               mask=idxr >= 0)
            idxn = lax.broadcast(HT * Wo + a, (L,)) + iota2
            plsc.store_scatter(otb, [idxn], nxb[...], mask=idxn < EDGE)
            pltpu.async_copy(
                otb,
                out_hbm.at[pl.ds(pl.multiple_of(w + 431 * Wo - a, 16), EDGE)],
                sot,
            )

            pltpu.make_async_copy(z_hbm.at[pl.ds(0, EDGE)], ohb, soh).wait()
            pltpu.async_copy(z_hbm.at[pl.ds(0, EDGE)], ohb, soh)
            pltpu.make_async_copy(z_hbm.at[pl.ds(0, EDGE)], otb, sot).wait()
            pltpu.async_copy(z_hbm.at[pl.ds(0, EDGE)], otb, sot)
            return carry

        lax.fori_loop(0, ch_per, do_plane, 0)

        for ob, osem in zip(out_bufs, out_sems):
            pltpu.make_async_copy(z_hbm, ob, osem).wait()
        pltpu.make_async_copy(z_hbm.at[pl.ds(0, EDGE)], ohb, soh).wait()
        pltpu.make_async_copy(z_hbm.at[pl.ds(0, EDGE)], otb, sot).wait()

    return zw(x3, ztile)


def kernel(x):
    B, C, H, W = x.shape
    Ho, Wo = 2 * H - 1, 2 * W - 1
    x3 = x.reshape(B * C, H, W)
    ztile = jnp.zeros((2 * RI * Wo,), jnp.float32)
    out = _zero_weave_sc(x3, ztile, BC=B * C, H=H, W=W)
    return out.reshape(B, C, Ho, Wo)


# speedup vs baseline: 1.5541x; 1.5541x over previous
import functools

import jax
import jax.numpy as jnp
from jax import lax
from jax.experimental import pallas as pl
from jax.experimental.pallas import tpu as pltpu
from jax.experimental.pallas import tpu_sc as plsc

L = 16
NC, NS = 2, 16
NW = NC * NS

RI = 32
RO = 2 * RI
NK = 7


def _zero_weave_sc(x3, ztile, *, BC, H, W):
    Ho, Wo = 2 * H - 1, 2 * W - 1
    ch_per = BC // NW

    mesh = plsc.VectorSubcoreMesh(
        core_axis_name="c", subcore_axis_name="s", num_cores=NC, num_subcores=NS
    )

    @functools.partial(
        pl.kernel,
        out_type=jax.ShapeDtypeStruct((BC, Ho, Wo), jnp.float32),
        mesh=mesh,
        scratch_types=[
            pltpu.VMEM((RI, W), jnp.float32),
            pltpu.VMEM((RO, Wo), jnp.float32),
        ],
        compiler_params=pltpu.CompilerParams(
            use_tc_tiling_on_sc=False, needs_layout_passes=False
        ),
    )
    def zw(x_hbm, z_hbm, out_hbm, ib, ob):
        wid = lax.axis_index("s") * NC + lax.axis_index("c")

        pltpu.sync_copy(z_hbm, ob)

        iota = lax.iota(jnp.int32, L)
        cvecs = [2 * (k * L + iota) for k in range(W // L)]

        def do_plane(ci, carry):
            ch = wid * ch_per + ci
            for k in range(NK):
                pltpu.sync_copy(x_hbm.at[ch, pl.ds(RI * k, RI), :], ib)

                def do_row(m, c2):
                    rvec = lax.broadcast(2 * m, (L,))
                    for kk in range(W // L):
                        plsc.store_scatter(
                            ob, [rvec, cvecs[kk]], ib[m, pl.ds(kk * L, L)]
                        )
                    return c2
                lax.fori_loop(0, RI, do_row, 0)

                rows = RO if k < NK - 1 else Ho - RO * (NK - 1)
                pltpu.sync_copy(
                    ob.at[pl.ds(0, rows)],
                    out_hbm.at[ch, pl.ds(RO * k, rows), :],
                )
            return carry

        lax.fori_loop(0, ch_per, do_plane, 0)

    return zw(x3, ztile)


def kernel(x):
    B, C, H, W = x.shape
    Ho, Wo = 2 * H - 1, 2 * W - 1
    x3 = x.reshape(B * C, H, W)
    ztile = jnp.zeros((RO, Wo), jnp.float32)
    out = _zero_weave_sc(x3, ztile, BC=B * C, H=H, W=W)
    return out.reshape(B, C, Ho, Wo)


# speedup vs baseline: 1.7951x; 1.1551x over previous
import functools

import jax
import jax.numpy as jnp
from jax import lax
from jax.experimental import pallas as pl
from jax.experimental.pallas import tpu as pltpu
from jax.experimental.pallas import tpu_sc as plsc

L = 16
NC, NS = 2, 16
NW = NC * NS

RI = 32
RO = 2 * RI
NK = 7


def _zero_weave_sc(x3, ztile, *, BC, H, W):
    Ho, Wo = 2 * H - 1, 2 * W - 1
    ch_per = BC // NW
    NG = ch_per * NK

    mesh = plsc.VectorSubcoreMesh(
        core_axis_name="c", subcore_axis_name="s", num_cores=NC, num_subcores=NS
    )

    @functools.partial(
        pl.kernel,
        out_type=jax.ShapeDtypeStruct((BC, Ho, Wo), jnp.float32),
        mesh=mesh,
        scratch_types=[
            pltpu.VMEM((RI, W), jnp.float32),
            pltpu.VMEM((RI, W), jnp.float32),
            pltpu.VMEM((RO, Wo), jnp.float32),
            pltpu.VMEM((RO, Wo), jnp.float32),
            pltpu.SemaphoreType.DMA,
            pltpu.SemaphoreType.DMA,
            pltpu.SemaphoreType.DMA,
            pltpu.SemaphoreType.DMA,
        ],
        compiler_params=pltpu.CompilerParams(
            use_tc_tiling_on_sc=False, needs_layout_passes=False
        ),
    )
    def zw(x_hbm, z_hbm, out_hbm, ib0, ib1, ob0, ob1, si0, si1, so0, so1):
        wid = lax.axis_index("s") * NC + lax.axis_index("c")

        in_bufs, in_sems = (ib0, ib1), (si0, si1)
        out_bufs, out_sems = (ob0, ob1), (so0, so1)

        def in_src(g):
            ci, k = divmod(g, NK)
            return x_hbm.at[wid * ch_per + ci, pl.ds(RI * k, RI), :]

        pltpu.async_copy(in_src(0), ib0, si0)
        pltpu.async_copy(in_src(1), ib1, si1)
        pltpu.async_copy(z_hbm, ob0, so0)
        pltpu.async_copy(z_hbm, ob1, so1)

        iota = lax.iota(jnp.int32, L)
        cvecs = [2 * (k * L + iota) for k in range(W // L)]

        rows_of = lambda g: RO if g % NK != NK - 1 else Ho - RO * (NK - 1)

        for g in range(NG):
            ci, k = divmod(g, NK)
            ch = wid * ch_per + ci
            q = g % 2
            ib, ob = in_bufs[q], out_bufs[q]

            pltpu.make_async_copy(in_src(g), ib, in_sems[q]).wait()
            if g < 2:
                pltpu.make_async_copy(z_hbm, ob, out_sems[q]).wait()
            else:
                gp = g - 2
                pltpu.make_async_copy(
                    ob.at[pl.ds(0, rows_of(gp))],
                    out_hbm.at[wid * ch_per + gp // NK,
                               pl.ds(RO * (gp % NK), rows_of(gp)), :],
                    out_sems[q],
                ).wait()

            def do_row(m, c2):
                rvec = lax.broadcast(2 * m, (L,))
                for kk in range(W // L):
                    plsc.store_scatter(
                        ob, [rvec, cvecs[kk]], ib[m, pl.ds(kk * L, L)]
                    )
                return c2
            lax.fori_loop(0, RI, do_row, 0)

            pltpu.async_copy(
                ob.at[pl.ds(0, rows_of(g))],
                out_hbm.at[ch, pl.ds(RO * k, rows_of(g)), :],
                out_sems[q],
            )
            if g + 2 < NG:
                pltpu.async_copy(in_src(g + 2), ib, in_sems[q])

        for gl in (NG - 2, NG - 1):
            q = gl % 2
            pltpu.make_async_copy(
                out_bufs[q].at[pl.ds(0, rows_of(gl))],
                out_hbm.at[wid * ch_per + gl // NK,
                           pl.ds(RO * (gl % NK), rows_of(gl)), :],
                out_sems[q],
            ).wait()

    return zw(x3, ztile)


def kernel(x):
    B, C, H, W = x.shape
    Ho, Wo = 2 * H - 1, 2 * W - 1
    x3 = x.reshape(B * C, H, W)
    ztile = jnp.zeros((RO, Wo), jnp.float32)
    out = _zero_weave_sc(x3, ztile, BC=B * C, H=H, W=W)
    return out.reshape(B, C, Ho, Wo)


# speedup vs baseline: 2.0545x; 1.1445x over previous
import functools

import jax
import jax.numpy as jnp
from jax import lax
from jax.experimental import pallas as pl
from jax.experimental.pallas import tpu as pltpu
from jax.experimental.pallas import tpu_sc as plsc

L = 16
NC, NS = 2, 16
NW = NC * NS

RI = 32
RO = 2 * RI
NK = 7


def _zero_weave_sc(x3, ztile, *, BC, NCH, H, W):
    Ho, Wo = 2 * H - 1, 2 * W - 1
    ch_per = BC // NW
    NG = ch_per * NK

    mesh = plsc.VectorSubcoreMesh(
        core_axis_name="c", subcore_axis_name="s", num_cores=NC, num_subcores=NS
    )

    @functools.partial(
        pl.kernel,
        out_type=jax.ShapeDtypeStruct((BC // NCH, Ho, NCH, Wo), jnp.float32),
        mesh=mesh,
        scratch_types=[
            pltpu.VMEM((RI, W), jnp.float32),
            pltpu.VMEM((RI, W), jnp.float32),
            pltpu.VMEM((RO, Wo), jnp.float32),
            pltpu.VMEM((RO, Wo), jnp.float32),
            pltpu.SemaphoreType.DMA,
            pltpu.SemaphoreType.DMA,
            pltpu.SemaphoreType.DMA,
            pltpu.SemaphoreType.DMA,
        ],
        compiler_params=pltpu.CompilerParams(
            use_tc_tiling_on_sc=False, needs_layout_passes=False
        ),
    )
    def zw(x_hbm, z_hbm, out_hbm, ib0, ib1, ob0, ob1, si0, si1, so0, so1):
        wid = lax.axis_index("s") * NC + lax.axis_index("c")

        in_bufs, in_sems = (ib0, ib1), (si0, si1)
        out_bufs, out_sems = (ob0, ob1), (so0, so1)

        def in_src(g):
            ci, k = divmod(g, NK)
            return x_hbm.at[wid * ch_per + ci, pl.ds(RI * k, RI), :]

        pltpu.async_copy(in_src(0), ib0, si0)
        pltpu.async_copy(in_src(1), ib1, si1)
        pltpu.async_copy(z_hbm, ob0, so0)
        pltpu.async_copy(z_hbm, ob1, so1)

        iota = lax.iota(jnp.int32, L)
        cvecs = [2 * (k * L + iota) for k in range(W // L)]

        rows_of = lambda g: RO if g % NK != NK - 1 else Ho - RO * (NK - 1)

        for g in range(NG):
            ci, k = divmod(g, NK)
            ch = wid * ch_per + ci
            q = g % 2
            ib, ob = in_bufs[q], out_bufs[q]

            pltpu.make_async_copy(in_src(g), ib, in_sems[q]).wait()
            if g < 2:
                pltpu.make_async_copy(z_hbm, ob, out_sems[q]).wait()
            else:
                gp = g - 2
                chp = wid * ch_per + gp // NK
                pltpu.make_async_copy(
                    ob.at[pl.ds(0, rows_of(gp))],
                    out_hbm.at[chp // NCH, pl.ds(RO * (gp % NK), rows_of(gp)),
                               chp % NCH, :],
                    out_sems[q],
                ).wait()

            def do_row(m, c2):
                rvec = lax.broadcast(2 * m, (L,))
                for kk in range(W // L):
                    plsc.store_scatter(
                        ob, [rvec, cvecs[kk]], ib[m, pl.ds(kk * L, L)]
                    )
                return c2
            lax.fori_loop(0, RI, do_row, 0)

            pltpu.async_copy(
                ob.at[pl.ds(0, rows_of(g))],
                out_hbm.at[ch // NCH, pl.ds(RO * k, rows_of(g)), ch % NCH, :],
                out_sems[q],
            )
            if g + 2 < NG:
                pltpu.async_copy(in_src(g + 2), ib, in_sems[q])

        for gl in (NG - 2, NG - 1):
            q = gl % 2
            chl = wid * ch_per + gl // NK
            pltpu.make_async_copy(
                out_bufs[q].at[pl.ds(0, rows_of(gl))],
                out_hbm.at[chl // NCH, pl.ds(RO * (gl % NK), rows_of(gl)),
                           chl % NCH, :],
                out_sems[q],
            ).wait()

    return zw(x3, ztile)


def kernel(x):
    B, C, H, W = x.shape
    x3 = x.reshape(B * C, H, W)
    ztile = jnp.zeros((RO, 2 * W - 1), jnp.float32)
    out = _zero_weave_sc(x3, ztile, BC=B * C, NCH=C, H=H, W=W)
    return jnp.transpose(out, (0, 2, 1, 3))
